# baseline (speedup 1.0000x reference)
import jax
import jax.numpy as jnp
from jax import lax
from jax.experimental import pallas as pl
from jax.experimental.pallas import tpu as pltpu
from jax.experimental.pallas import tpu_sc as plsc

N = 100000
D = 128
E = 500000
L = 100000

NP = 104448
RB = 6144
RPAD = RB + 16
NBUK = 17
BPC = 9
TPB = RB // 16

EPT = 32768
EPAD = 16 * EPT
ECH = 2048
NCH = EPT // ECH
GCH = 256
LCAP = ECH + GCH + 16
TRASH = ECH + GCH

HPT = EPAD // 32
HCH = 2048

LPT = 3328
LPAD = 32 * LPT
LCH = 256


def _sc_mesh():
    return plsc.VectorSubcoreMesh(core_axis_name="c", subcore_axis_name="s")


def _prefix_incl(x, lane):
    for dd in (1, 2, 4, 8):
        sh = jnp.take(x, jnp.maximum(lane - dd, 0))
        x = x + jnp.where(lane >= dd, sh, 0)
    return x


def _seg_body(x_hbm, src_hbm, dst_hbm, zrows_hbm, agg_hbm,
              sstg, dstg, srcl, dstl, srow, drow, rowbuf, acc):
    c = lax.axis_index("c")
    s = lax.axis_index("s")
    lane = lax.broadcasted_iota(jnp.int32, (16,), 0)

    def bucket_body(p, _):
        bid = BPC * c + p - c
        lo = pl.multiple_of(bid * RB, 16)
        my0 = pl.multiple_of(s * TPB, 16)
        run = bid < NBUK

        @pl.when(run)
        def _zero():
            pltpu.sync_copy(zrows_hbm, acc.at[pl.ds(my0, TPB)])

        plsc.subcore_barrier()

        def chunk_body(k, _, lo=lo):
            ebase = pl.multiple_of(s * EPT + k * ECH, ECH)
            pltpu.sync_copy(src_hbm.at[pl.ds(ebase, ECH)], sstg)
            pltpu.sync_copy(dst_hbm.at[pl.ds(ebase, ECH)], dstg)

            def scan_body(i, off, lo=lo):
                o16 = pl.multiple_of(i * 16, 16)
                dv = dstg[pl.ds(o16, 16)]
                sv = sstg[pl.ds(o16, 16)]
                rel = dv - lo
                m = (rel >= 0) & (rel < RB)
                x = _prefix_incl(m.astype(jnp.int32), lane)
                pos = jnp.where(m, off + x - 1, TRASH + lane)
                plsc.store_scatter(srcl, [pos], sv)
                plsc.store_scatter(dstl, [pos], rel)
                return off + x[15]

            n = lax.fori_loop(0, ECH // 16, scan_body, jnp.int32(0))

            dummy_d = jnp.full((16,), RB, jnp.int32)
            dummy_s = jnp.zeros((16,), jnp.int32)

            def pad_body(j, _):
                idxp = n + j * 16 + lane
                plsc.store_scatter(srcl, [idxp], dummy_s)
                plsc.store_scatter(dstl, [idxp], dummy_d)
                return 0

            lax.fori_loop(0, GCH // 16, pad_body, 0)

            ndr = (n + GCH - 1) // GCH

            def drain_body(g, _):
                def cp(j, _):
                    o = pl.multiple_of(g * GCH + j * 16, 16)
                    oj = pl.multiple_of(j * 16, 16)
                    srow[pl.ds(oj, 16)] = srcl[pl.ds(o, 16)]
                    drow[pl.ds(oj, 16)] = dstl[pl.ds(o, 16)]
                    return 0

                lax.fori_loop(0, GCH // 16, cp, 0)
                pltpu.sync_copy(x_hbm.at[srow], rowbuf)
                pltpu.sync_copy(rowbuf, acc.at[drow], add=True)
                return 0

            lax.fori_loop(0, ndr, drain_body, 0)
            return 0

        @pl.when(run)
        def _scan():
            lax.fori_loop(0, NCH, chunk_body, 0)

        plsc.subcore_barrier()

        @pl.when(run)
        def _drain():
            out0 = pl.multiple_of(lo + s * TPB, 16)
            pltpu.sync_copy(acc.at[pl.ds(my0, TPB)],
                            agg_hbm.at[pl.ds(out0, TPB)])

        plsc.subcore_barrier()
        return 0

    lax.fori_loop(0, BPC, bucket_body, 0)


def _seg(x, src, dst, zrows):
    kfn = pl.kernel(
        _seg_body,
        out_type=jax.ShapeDtypeStruct((NP, D), jnp.float32),
        mesh=_sc_mesh(),
        compiler_params=pltpu.CompilerParams(needs_layout_passes=False),
        scratch_types=[
            pltpu.VMEM((ECH,), jnp.int32),
            pltpu.VMEM((ECH,), jnp.int32),
            pltpu.VMEM((LCAP,), jnp.int32),
            pltpu.VMEM((LCAP,), jnp.int32),
            pltpu.VMEM((GCH,), jnp.int32),
            pltpu.VMEM((GCH,), jnp.int32),
            pltpu.VMEM((GCH, D), jnp.float32),
            pltpu.VMEM_SHARED((RPAD, D), jnp.float32),
        ],
    )
    return kfn(x, src, dst, zrows)


def _cnt_body(dst_hbm, zhist_hbm, out_hbm, dstg, hist):
    c = lax.axis_index("c")
    s = lax.axis_index("s")
    w = c * 16 + s
    lane = lax.broadcasted_iota(jnp.int32, (16,), 0)
    pltpu.sync_copy(zhist_hbm, hist)

    def chunk_body(k, _):
        ebase = pl.multiple_of(w * HPT + k * HCH, HCH)
        pltpu.sync_copy(dst_hbm.at[pl.ds(ebase, HCH)], dstg)

        def scan_body(i, _):
            o16 = pl.multiple_of(i * 16, 16)
            dv = dstg[pl.ds(o16, 16)]
            dv = jnp.minimum(dv, NP)
            count = jnp.ones((16,), jnp.int32)
            earlier = jnp.zeros((16,), jnp.bool_)
            for dd in range(1, 16):
                rot = jnp.take(dv, (lane + dd) & 15)
                e = dv == rot
                count = count + e.astype(jnp.int32)
                lt = ((lane + dd) & 15) < lane
                earlier = earlier | (e & lt)
            first = ~earlier
            addr = jnp.where(first, dv, NP + lane)
            val = jnp.where(first, count.astype(jnp.float32), 0.0)
            plsc.addupdate_scatter(hist, [addr], val)
            return 0

        lax.fori_loop(0, HCH // 16, scan_body, 0)
        return 0

    lax.fori_loop(0, HPT // HCH, chunk_body, 0)
    pltpu.sync_copy(hist.at[pl.ds(0, NP)], out_hbm.at[w])


def _cnt(dst, zhist):
    kfn = pl.kernel(
        _cnt_body,
        out_type=jax.ShapeDtypeStruct((32, NP), jnp.float32),
        mesh=_sc_mesh(),
        compiler_params=pltpu.CompilerParams(needs_layout_passes=False),
        scratch_types=[
            pltpu.VMEM((HCH,), jnp.int32),
            pltpu.VMEM((NP + 16,), jnp.float32),
        ],
    )
    return kfn(dst, zhist)


def _merge_body(p_ref, out_ref):
    out_ref[...] = jnp.sum(p_ref[...], axis=0, keepdims=True)


def _merge(partials):
    mb = 2048
    return pl.pallas_call(
        _merge_body,
        grid=(NP // mb,),
        in_specs=[pl.BlockSpec((32, mb), lambda i: (0, i))],
        out_specs=pl.BlockSpec((1, mb), lambda i: (0, i)),
        out_shape=jax.ShapeDtypeStruct((1, NP), jnp.float32),
    )(partials)


BLK = 512


def _dense1_body(agg_ref, cnt_ref, x_ref, wl_ref, wr_ref, b_ref, out_ref):
    mean = agg_ref[...] / jnp.maximum(cnt_ref[...], 1.0)
    z = lax.dot_general(mean, wl_ref[...], (((1,), (1,)), ((), ())),
                        preferred_element_type=jnp.float32)
    z = z + lax.dot_general(x_ref[...], wr_ref[...], (((1,), (1,)), ((), ())),
                            preferred_element_type=jnp.float32)
    z = z + b_ref[...]
    nrm = jnp.sqrt(jnp.sum(z * z, axis=1, keepdims=True))
    h = z / jnp.maximum(nrm, 1e-12)
    out_ref[...] = jnp.where(h >= 0, h, 0.01 * h)


def _dense2_body(agg_ref, cnt_ref, x_ref, wl_ref, wr_ref, b_ref, out_ref):
    mean = agg_ref[...] / jnp.maximum(cnt_ref[...], 1.0)
    z = lax.dot_general(mean, wl_ref[...], (((1,), (1,)), ((), ())),
                        preferred_element_type=jnp.float32)
    z = z + lax.dot_general(x_ref[...], wr_ref[...], (((1,), (1,)), ((), ())),
                            preferred_element_type=jnp.float32)
    z = z + b_ref[...]
    nrm = jnp.sqrt(jnp.sum(z * z, axis=1, keepdims=True))
    out_ref[...] = x_ref[...] + z / jnp.maximum(nrm, 1e-12)


def _dense(body, agg, cnt, x, wl, wr, b2):
    return pl.pallas_call(
        body,
        grid=(NP // BLK,),
        in_specs=[
            pl.BlockSpec((BLK, D), lambda i: (i, 0)),
            pl.BlockSpec((BLK, 1), lambda i: (i, 0)),
            pl.BlockSpec((BLK, D), lambda i: (i, 0)),
            pl.BlockSpec((D, D), lambda i: (0, 0)),
            pl.BlockSpec((D, D), lambda i: (0, 0)),
            pl.BlockSpec((1, D), lambda i: (0, 0)),
        ],
        out_specs=pl.BlockSpec((BLK, D), lambda i: (i, 0)),
        out_shape=jax.ShapeDtypeStruct((NP, D), jnp.float32),
    )(agg, cnt, x, wl, wr, b2)


def _cls_body(hs_hbm, ht_hbm, i0_hbm, i1_hbm, out_hbm, i0v, i1v, esv, etv, ov):
    c = lax.axis_index("c")
    s = lax.axis_index("s")
    base = (c * 16 + s) * LPT
    lane = lax.broadcasted_iota(jnp.int32, (16,), 0)

    def chunk(k, _):
        off = pl.multiple_of(base + k * LCH, LCH)
        pltpu.sync_copy(i0_hbm.at[pl.ds(off, LCH)], i0v)
        pltpu.sync_copy(i1_hbm.at[pl.ds(off, LCH)], i1v)
        pltpu.sync_copy(hs_hbm.at[i0v], esv)
        pltpu.sync_copy(ht_hbm.at[i1v], etv)

        def grp(g, _):
            ovec = jnp.zeros((16,), jnp.float32)
            for j in range(16):
                r = g * 16 + j
                acc = jnp.zeros((16,), jnp.float32)
                for q in range(D // 16):
                    acc = acc + (esv[r, pl.ds(q * 16, 16)]
                                 * etv[r, pl.ds(q * 16, 16)])
                for dd in (1, 2, 4, 8):
                    acc = acc + jnp.take(acc, lane ^ dd)
                ovec = jnp.where(lane == j, acc, ovec)
            og = pl.multiple_of(g * 16, 16)
            ov[pl.ds(og, 16)] = ovec
            return 0

        lax.fori_loop(0, LCH // 16, grp, 0)
        pltpu.sync_copy(ov, out_hbm.at[pl.ds(off, LCH)])
        return 0

    lax.fori_loop(0, LPT // LCH, chunk, 0)


def _cls(hs, ht, i0, i1):
    kfn = pl.kernel(
        _cls_body,
        out_type=jax.ShapeDtypeStruct((LPAD,), jnp.float32),
        mesh=_sc_mesh(),
        compiler_params=pltpu.CompilerParams(needs_layout_passes=False),
        scratch_types=[
            pltpu.VMEM((LCH,), jnp.int32),
            pltpu.VMEM((LCH,), jnp.int32),
            pltpu.VMEM((LCH, D), jnp.float32),
            pltpu.VMEM((LCH, D), jnp.float32),
            pltpu.VMEM((LCH,), jnp.float32),
        ],
    )
    return kfn(hs, ht, i0, i1)


def kernel(source_node_id, target_node_id, edge_index_binds, edge_index_rev,
           edge_label_index, source_emb, target_emb,
           W1b_l, W1b_r, W1r_l, W1r_r, W2b_l, W2b_r, W2r_l, W2r_r,
           b1b, b1r, b2b, b2r):
    f32 = jnp.float32
    npad = NP - N
    xs = jnp.concatenate([source_emb, jnp.zeros((npad, D), f32)], axis=0)
    xt = jnp.concatenate([target_emb, jnp.zeros((npad, D), f32)], axis=0)

    epad = EPAD - E
    pad_dst = jnp.full((epad,), 1 << 20, jnp.int32)
    pad_zero = jnp.zeros((epad,), jnp.int32)
    src_b = jnp.concatenate([edge_index_binds[0], pad_zero])
    dst_b = jnp.concatenate([edge_index_binds[1], pad_dst])
    src_r = jnp.concatenate([edge_index_rev[0], pad_zero])
    dst_r = jnp.concatenate([edge_index_rev[1], pad_dst])

    lpad = LPAD - L
    lz = jnp.zeros((lpad,), jnp.int32)
    eli0 = jnp.concatenate([edge_label_index[0], lz])
    eli1 = jnp.concatenate([edge_label_index[1], lz])

    zrows = jnp.zeros((TPB, D), f32)
    zhist = jnp.zeros((NP + 16,), f32)

    cnt_b = _merge(_cnt(dst_b, zhist)).reshape(NP, 1)
    cnt_r = _merge(_cnt(dst_r, zhist)).reshape(NP, 1)
    agg1t = _seg(xs, src_b, dst_b, zrows)
    agg1s = _seg(xt, src_r, dst_r, zrows)
    h1t = _dense(_dense1_body, agg1t, cnt_b, xt, W1b_l, W1b_r, b1b[None, :])
    h1s = _dense(_dense1_body, agg1s, cnt_r, xs, W1r_l, W1r_r, b1r[None, :])
    agg2t = _seg(h1s, src_b, dst_b, zrows)
    agg2s = _seg(h1t, src_r, dst_r, zrows)
    h3t = _dense(_dense2_body, agg2t, cnt_b, h1t, W2b_l, W2b_r, b2b[None, :])
    h3s = _dense(_dense2_body, agg2s, cnt_r, h1s, W2r_l, W2r_r, b2r[None, :])

    dots = _cls(h3s, h3t, eli0, eli1)
    return dots[:L]
